# Initial kernel scaffold; baseline (speedup 1.0000x reference)
import jax
import jax.numpy as jnp
from jax.experimental import pallas as pl


def kernel(feat, user_ids, item_ids, edge_src, edge_dst, W_src, b_src, W_dst, b_dst):
    raise NotImplementedError("write your pallas kernel here")


# speedup vs baseline: 1.0965x; 1.0965x over previous
import functools

import jax
import jax.numpy as jnp
from jax import lax
from jax.experimental import pallas as pl
from jax.experimental.pallas import tpu as pltpu
from jax.experimental.pallas import tpu_sc as plsc

F = 256
N_USERS = 5000
N_ITEMS = 5000
N_EDGES = 160000
NC, NS = 2, 16
NW = NC * NS
E_PAD = 163840
N_PAD = 5120


def _pick_chunk(n):
    for c in (128, 120, 112, 104, 96, 88, 80, 72, 64, 56, 48, 40, 32, 24, 16, 8):
        if n % c == 0:
            return c
    raise ValueError(n)


def _sc_gather_rows(table, idx):
    B = idx.shape[0]
    V, D = table.shape
    b_per_w = B // NW
    C = _pick_chunk(b_per_w)
    n_iter = b_per_w // C
    mesh = plsc.VectorSubcoreMesh(core_axis_name="c", subcore_axis_name="s")

    @functools.partial(
        pl.kernel,
        mesh=mesh,
        out_type=jax.ShapeDtypeStruct((B, D), jnp.float32),
        scratch_types=[
            pltpu.VMEM((C,), jnp.int32),
            pltpu.VMEM((C, D), jnp.float32),
            pltpu.SemaphoreType.DMA,
        ],
    )
    def gather_k(table_hbm, idx_hbm, out_hbm, idx_v, rows_v, sem):
        wid = lax.axis_index("s") * NC + lax.axis_index("c")

        def body(i, carry):
            base = wid * b_per_w + i * C
            pltpu.sync_copy(idx_hbm.at[pl.ds(base, C)], idx_v)
            pltpu.async_copy(table_hbm.at[idx_v], rows_v, sem).wait()
            pltpu.sync_copy(rows_v, out_hbm.at[pl.ds(base, C)])
            return carry

        lax.fori_loop(0, n_iter, body, 0)

    return gather_k(table, idx)


def _sc_segment_sum(m_items, m_users, dst_idx, src_idx, zeros):
    per_sub = E_PAD // NS
    C = 128
    n_iter = per_sub // C
    rows_per_sub = N_PAD // NS
    mesh = plsc.VectorSubcoreMesh(core_axis_name="c", subcore_axis_name="s")

    @functools.partial(
        pl.kernel,
        mesh=mesh,
        out_type=(
            jax.ShapeDtypeStruct((N_PAD, F), jnp.float32),
            jax.ShapeDtypeStruct((N_PAD, F), jnp.float32),
        ),
        scratch_types=[
            pltpu.VMEM((C,), jnp.int32),
            pltpu.VMEM((C, F), jnp.float32),
            pltpu.VMEM((rows_per_sub, F), jnp.float32),
        ],
    )
    def seg_k(mi_hbm, mu_hbm, di_hbm, si_hbm, z_hbm,
              item_out, user_out, idx_v, rows_v, z_v):
        cid = lax.axis_index("c")
        sid = lax.axis_index("s")

        def run(m_hbm, i_hbm, out_hbm):
            r0 = sid * rows_per_sub
            pltpu.sync_copy(z_hbm.at[pl.ds(r0, rows_per_sub)], z_v)
            pltpu.sync_copy(z_v, out_hbm.at[pl.ds(r0, rows_per_sub)])
            plsc.subcore_barrier()

            def body(i, carry):
                base = sid * per_sub + i * C
                pltpu.sync_copy(i_hbm.at[pl.ds(base, C)], idx_v)
                pltpu.sync_copy(m_hbm.at[pl.ds(base, C)], rows_v)
                pltpu.sync_copy(rows_v, out_hbm.at[idx_v], add=True)
                return carry

            lax.fori_loop(0, n_iter, body, 0)

        @pl.when(cid == 0)
        def _():
            run(mi_hbm, di_hbm, item_out)

        @pl.when(cid == 1)
        def _():
            run(mu_hbm, si_hbm, user_out)

    return seg_k(m_items, m_users, dst_idx, src_idx, zeros)


_BLK = 256
_NBLK = E_PAD // _BLK


def _edge_dot_kernel(hs_ref, hd_ref, e_ref):
    i = pl.program_id(0)
    prod = hs_ref[...] * hd_ref[...]
    s = jnp.sum(prod, axis=1, keepdims=True) * (1.0 / 16.0)
    row = i * _BLK + lax.broadcasted_iota(jnp.int32, (_BLK, 1), 0)
    s = jnp.where(row < N_EDGES, s, -1e30)
    e_ref[...] = s.reshape(1, 1, _BLK)


def _tc_edge_dots(hs_e, hd_e):
    return pl.pallas_call(
        _edge_dot_kernel,
        grid=(_NBLK,),
        in_specs=[
            pl.BlockSpec((_BLK, F), lambda i: (i, 0)),
            pl.BlockSpec((_BLK, F), lambda i: (i, 0)),
        ],
        out_specs=pl.BlockSpec((1, 1, _BLK), lambda i: (i, 0, 0)),
        out_shape=jax.ShapeDtypeStruct((_NBLK, 1, _BLK), jnp.float32),
    )(hs_e, hd_e)


def _softmax_kernel(e_ref, a_ref):
    e = e_ref[...]
    m = jnp.max(e)
    ex = jnp.exp(e - m)
    a_ref[...] = ex * (1.0 / jnp.sum(ex))


def _tc_softmax(e3):
    return pl.pallas_call(
        _softmax_kernel,
        out_shape=jax.ShapeDtypeStruct((_NBLK, 1, _BLK), jnp.float32),
    )(e3)


def _msg_kernel(hs_ref, hd_ref, a_ref, ws_ref, bs_ref, wd_ref, bd_ref,
                mi_ref, mu_ref):
    alpha = a_ref[...].reshape(_BLK, 1)
    fs = jnp.maximum(
        lax.dot_general(hs_ref[...], ws_ref[...], (((1,), (0,)), ((), ())),
                        precision=lax.Precision.HIGHEST,
                        preferred_element_type=jnp.float32) + bs_ref[...], 0.0)
    mi_ref[...] = fs * alpha
    fd = jnp.maximum(
        lax.dot_general(hd_ref[...], wd_ref[...], (((1,), (0,)), ((), ())),
                        precision=lax.Precision.HIGHEST,
                        preferred_element_type=jnp.float32) + bd_ref[...], 0.0)
    mu_ref[...] = fd * alpha


def _tc_messages(hs_e, hd_e, alpha3, WsT, bs2, WdT, bd2):
    return pl.pallas_call(
        _msg_kernel,
        grid=(_NBLK,),
        in_specs=[
            pl.BlockSpec((_BLK, F), lambda i: (i, 0)),
            pl.BlockSpec((_BLK, F), lambda i: (i, 0)),
            pl.BlockSpec((1, 1, _BLK), lambda i: (i, 0, 0)),
            pl.BlockSpec((F, F), lambda i: (0, 0)),
            pl.BlockSpec((1, F), lambda i: (0, 0)),
            pl.BlockSpec((F, F), lambda i: (0, 0)),
            pl.BlockSpec((1, F), lambda i: (0, 0)),
        ],
        out_specs=[
            pl.BlockSpec((_BLK, F), lambda i: (i, 0)),
            pl.BlockSpec((_BLK, F), lambda i: (i, 0)),
        ],
        out_shape=[
            jax.ShapeDtypeStruct((E_PAD, F), jnp.float32),
            jax.ShapeDtypeStruct((E_PAD, F), jnp.float32),
        ],
    )(hs_e, hd_e, alpha3, WsT, bs2, WdT, bd2)


def kernel(feat, user_ids, item_ids, edge_src, edge_dst,
           W_src, b_src, W_dst, b_dst):
    uid_p = jnp.pad(user_ids.astype(jnp.int32), (0, N_PAD - N_USERS))
    iid_p = jnp.pad(item_ids.astype(jnp.int32), (0, N_PAD - N_ITEMS))
    es_p = jnp.pad(edge_src.astype(jnp.int32), (0, E_PAD - N_EDGES))
    ed_p = jnp.pad(edge_dst.astype(jnp.int32), (0, E_PAD - N_EDGES))

    h_src = _sc_gather_rows(feat, uid_p)
    h_dst = _sc_gather_rows(feat, iid_p)

    hs_e = _sc_gather_rows(h_src, es_p)
    hd_e = _sc_gather_rows(h_dst, ed_p)

    e3 = _tc_edge_dots(hs_e, hd_e)
    alpha3 = _tc_softmax(e3)

    m_items, m_users = _tc_messages(
        hs_e, hd_e, alpha3,
        W_src.T, b_src.reshape(1, F),
        W_dst.T, b_dst.reshape(1, F))

    zeros = jnp.zeros((N_PAD, F), jnp.float32)
    item_acc = jax.ops.segment_sum(m_items, ed_p, num_segments=N_PAD)
    user_acc = jax.ops.segment_sum(m_users, es_p, num_segments=N_PAD)

    return jnp.concatenate([user_acc[:N_USERS], item_acc[:N_ITEMS]], axis=0)
